# baseline (speedup 1.0000x reference)
import functools

import jax
import jax.numpy as jnp
from jax.experimental import pallas as pl


def _pick_rows(h):
    for r in (28, 16, 8, 4):
        if h % r == 0:
            return r
    return h


def _fused_gcn_kernel(main_ref, top_ref, bot_ref, wz_ref, w2t_ref,
                      b1_ref, b2_ref, lin_b_ref, out_ref,
                      *, rows, halo, height, width, wp, hid):
    i = pl.program_id(1)
    mh = (rows + 2 * halo) * wp

    xh = jnp.concatenate([top_ref[0], main_ref[0], bot_ref[0]], axis=1)

    m = jax.lax.broadcasted_iota(jnp.int32, (1, mh), 1)
    row_local = m // wp
    col = m - row_local * wp
    row = i * rows - halo + row_local
    deg = (5.0
           - (col == 0).astype(jnp.float32)
           - (col == width - 1).astype(jnp.float32)
           - (row == 0).astype(jnp.float32)
           - (row == height - 1).astype(jnp.float32))
    valid = (row >= 0) & (row < height) & (col < width)
    dinv = jnp.where(valid, jax.lax.rsqrt(deg), 0.0)

    def stencil(y):
        f = y.shape[0]
        zw = jnp.zeros((f, wp), y.dtype)
        z1 = jnp.zeros((f, 1), y.dtype)
        up = jnp.concatenate([zw, y[:, :-wp]], axis=1)
        dn = jnp.concatenate([y[:, wp:], zw], axis=1)
        lf = jnp.concatenate([z1, y[:, :-1]], axis=1)
        rt = jnp.concatenate([y[:, 1:], z1], axis=1)
        return y + up + dn + lf + rt

    prec = jax.lax.Precision.DEFAULT

    zfull = jnp.dot(wz_ref[...], xh, precision=prec,
                    preferred_element_type=jnp.float32)
    z1 = zfull[:hid]
    h1 = jax.nn.relu(dinv * stencil(z1 * dinv) + b1_ref[...])
    t = h1 + xh
    z2 = jnp.dot(w2t_ref[...], t, precision=prec,
                 preferred_element_type=jnp.float32)
    h2 = dinv * stencil(z2 * dinv) + b2_ref[...]

    c0 = halo * wp
    c1 = c0 + rows * wp
    origin = zfull[hid:, c0:c1] + lin_b_ref[...]
    out_ref[0] = h2[:, c0:c1] * origin


def kernel(x, lin_w, lin_b, w1, b1, w2, b2):
    bsz, c, h, w = x.shape
    hid = w1.shape[1]
    out_f = w2.shape[1]
    wp = -(-w // 128) * 128
    if wp == w:
        wp += 128
    np_ = h * wp
    rows = _pick_rows(h)
    halo = 2
    nblk = h // rows
    hb = halo * wp

    xp = jnp.pad(x, ((0, 0), (0, 0), (0, 0), (0, wp - w)))
    x3 = xp.reshape(bsz, c, np_)
    wz = jnp.concatenate([w1.T, lin_w], axis=0)
    w2t = w2.T
    b1c = b1.reshape(hid, 1)
    b2c = b2.reshape(out_f, 1)
    lin_bc = lin_b.reshape(out_f, 1)

    halo_blocks = np_ // hb
    full = lambda a: pl.BlockSpec(a.shape, lambda b, i: (0,) * a.ndim)

    grid_kernel = functools.partial(
        _fused_gcn_kernel, rows=rows, halo=halo, height=h, width=w,
        wp=wp, hid=hid)

    out3 = pl.pallas_call(
        grid_kernel,
        grid=(bsz, nblk),
        in_specs=[
            pl.BlockSpec((1, c, rows * wp), lambda b, i: (b, 0, i)),
            pl.BlockSpec((1, c, hb),
                         lambda b, i: (b, 0, jnp.maximum(i * (rows // halo) - 1, 0))),
            pl.BlockSpec((1, c, hb),
                         lambda b, i: (b, 0, jnp.minimum((i + 1) * (rows // halo),
                                                         halo_blocks - 1))),
            full(wz), full(w2t), full(b1c), full(b2c), full(lin_bc),
        ],
        out_specs=pl.BlockSpec((1, out_f, rows * wp), lambda b, i: (b, 0, i)),
        out_shape=jax.ShapeDtypeStruct((bsz, out_f, np_), jnp.float32),
    )(x3, x3, x3, wz, w2t, b1c, b2c, lin_bc)

    return out3.reshape(bsz, out_f, h, wp)[:, :, :, :w]


# speedup vs baseline: 1.0774x; 1.0774x over previous
import functools

import jax
import jax.numpy as jnp
from jax.experimental import pallas as pl
from jax.experimental.pallas import tpu as pltpu


def _pick_rows(h):
    for r in (32, 56, 16, 8):
        if h % r == 0 and r % 8 == 0:
            return r
    return h


def _fused_gcn_kernel(main_ref, bot_ref, wz_ref, w2t_ref,
                      b1_ref, b2_ref, lin_b_ref, out_ref, prev_ref,
                      *, rows, halo, height, width, wp, hid):
    i = pl.program_id(1)
    mh = (rows + 2 * halo) * wp

    def flat(piece):
        cc, rr, ww = piece.shape
        piece = jnp.concatenate(
            [piece, jnp.zeros((cc, rr, wp - ww), piece.dtype)], axis=2)
        return piece.reshape(cc, rr * wp)

    top = jnp.where(i > 0, flat(prev_ref[...]), 0.0)
    xh = jnp.concatenate(
        [top, flat(main_ref[0]), flat(bot_ref[0][:, :halo, :])], axis=1)
    prev_ref[...] = main_ref[0][:, rows - halo:, :]

    m = jax.lax.broadcasted_iota(jnp.int32, (1, mh), 1)
    row_local = m // wp
    col = m - row_local * wp
    row = i * rows - halo + row_local
    deg = (5.0
           - (col == 0).astype(jnp.float32)
           - (col == width - 1).astype(jnp.float32)
           - (row == 0).astype(jnp.float32)
           - (row == height - 1).astype(jnp.float32))
    valid = (row >= 0) & (row < height) & (col < width)
    dinv = jnp.where(valid, jax.lax.rsqrt(deg), 0.0)

    def stencil(y):
        f = y.shape[0]
        zw = jnp.zeros((f, halo * wp), y.dtype)
        z1 = jnp.zeros((f, 1), y.dtype)
        up = jnp.concatenate([zw[:, :wp], y[:, :-wp]], axis=1)
        dn = jnp.concatenate([y[:, wp:], zw[:, :wp]], axis=1)
        lf = jnp.concatenate([z1, y[:, :-1]], axis=1)
        rt = jnp.concatenate([y[:, 1:], z1], axis=1)
        return y + up + dn + lf + rt

    prec = jax.lax.Precision.DEFAULT

    zfull = jnp.dot(wz_ref[...], xh, precision=prec,
                    preferred_element_type=jnp.float32)
    z1 = zfull[:hid]
    h1 = jax.nn.relu(dinv * stencil(z1 * dinv) + b1_ref[...])
    t = h1 + xh
    z2 = jnp.dot(w2t_ref[...], t, precision=prec,
                 preferred_element_type=jnp.float32)
    h2 = dinv * stencil(z2 * dinv) + b2_ref[...]

    c0 = halo * wp
    c1 = c0 + rows * wp
    origin = zfull[hid:, c0:c1] + lin_b_ref[...]
    res = (h2[:, c0:c1] * origin).reshape(-1, rows, wp)
    out_ref[0] = res[:, :, :width]


def kernel(x, lin_w, lin_b, w1, b1, w2, b2):
    bsz, c, h, w = x.shape
    hid = w1.shape[1]
    out_f = w2.shape[1]
    wp = -(-w // 128) * 128
    if wp == w:
        wp += 128
    rows = _pick_rows(h)
    halo = 2
    fetch = 8
    nblk = h // rows

    wz = jnp.concatenate([w1.T, lin_w], axis=0)
    w2t = w2.T
    b1c = b1.reshape(hid, 1)
    b2c = b2.reshape(out_f, 1)
    lin_bc = lin_b.reshape(out_f, 1)

    fetch_blocks = h // fetch
    full = lambda a: pl.BlockSpec(a.shape, lambda b, i: (0,) * a.ndim)

    grid_kernel = functools.partial(
        _fused_gcn_kernel, rows=rows, halo=halo, height=h, width=w,
        wp=wp, hid=hid)

    return pl.pallas_call(
        grid_kernel,
        grid=(bsz, nblk),
        in_specs=[
            pl.BlockSpec((1, c, rows, w), lambda b, i: (b, 0, i, 0)),
            pl.BlockSpec((1, c, fetch, w),
                         lambda b, i: (b, 0, jnp.minimum((i + 1) * (rows // fetch),
                                                         fetch_blocks - 1), 0)),
            full(wz), full(w2t), full(b1c), full(b2c), full(lin_bc),
        ],
        out_specs=pl.BlockSpec((1, out_f, rows, w), lambda b, i: (b, 0, i, 0)),
        out_shape=jax.ShapeDtypeStruct((bsz, out_f, h, w), jnp.float32),
        scratch_shapes=[pltpu.VMEM((c, halo, w), jnp.float32)],
        compiler_params=pltpu.CompilerParams(
            vmem_limit_bytes=100 * 1024 * 1024),
    )(x, x, wz, w2t, b1c, b2c, lin_bc)
